# baseline (speedup 1.0000x reference)
import jax
import jax.numpy as jnp
from jax.experimental import pallas as pl

_L = 2048
_N = 32
_X, _Y = 8, 4


def _body(w_ref, h_ref, o_ref):
    p = jnp.sum(w_ref[...] * h_ref[...], axis=0, keepdims=True)
    o_ref[...] = (p[:, 0:32] + p[:, 32:64]) + (p[:, 64:96] + p[:, 96:128])


@jax.jit
def _run(w2d, h2d):
    return pl.pallas_call(
        _body,
        out_shape=jax.ShapeDtypeStruct((1, _N), jnp.float32),
    )(w2d, h2d)


def kernel(x, adj, W_att, a_att, W_out):
    h2d = jnp.reshape(x, (_L // 4, 128))
    w2d = jnp.reshape(W_out, (_L // 4, 128))
    return jnp.reshape(_run(w2d, h2d), (_X, _Y))


# speedup vs baseline: 3.6433x; 3.6433x over previous
import jax
import jax.numpy as jnp
from jax.experimental import pallas as pl
from jax.experimental.pallas import tpu as pltpu

_L = 2048
_N = 32
_X, _Y = 8, 4
_BLK = 512


def _body(w_ref, h_ref, o_ref):
    i = pl.program_id(0)

    @pl.when(i == 0)
    def _():
        o_ref[...] = jnp.zeros_like(o_ref)

    o_ref[...] += jnp.sum(w_ref[...] * h_ref[...], axis=0, keepdims=True)


@jax.jit
def _run(w2d, h2d):
    return pl.pallas_call(
        _body,
        grid=(_L // _BLK,),
        in_specs=[
            pl.BlockSpec((_BLK, _N), lambda i: (i, 0)),
            pl.BlockSpec((_BLK, _N), lambda i: (i, 0)),
        ],
        out_specs=pl.BlockSpec((1, _N), lambda i: (0, 0)),
        out_shape=jax.ShapeDtypeStruct((1, _N), jnp.float32),
        compiler_params=pltpu.CompilerParams(
            dimension_semantics=("arbitrary",)
        ),
    )(w2d, h2d)


def kernel(x, adj, W_att, a_att, W_out):
    h2d = jnp.reshape(x, (_L, _N))
    w2d = jnp.reshape(W_out, (_L, _N))
    return jnp.reshape(_run(w2d, h2d), (_X, _Y))


# speedup vs baseline: 5.4340x; 1.4915x over previous
import jax
import jax.numpy as jnp
from jax.experimental import pallas as pl

_L = 2048
_N = 32
_X, _Y = 8, 4


def _body(w_ref, h_ref, o_ref):
    o_ref[...] = jnp.sum(w_ref[...] * h_ref[...], axis=1, keepdims=True)


@jax.jit
def _run(wT, hT):
    return pl.pallas_call(
        _body,
        out_shape=jax.ShapeDtypeStruct((_N, 1), jnp.float32),
    )(wT, hT)


def kernel(x, adj, W_att, a_att, W_out):
    hT = jnp.reshape(jnp.transpose(x[0], (1, 2, 0)), (_N, _L))
    wT = jnp.reshape(jnp.transpose(W_out, (1, 2, 0)), (_N, _L))
    return jnp.reshape(_run(wT, hT), (_X, _Y))


# speedup vs baseline: 10.3966x; 1.9132x over previous
import jax
import jax.numpy as jnp
from jax.experimental import pallas as pl

_L = 2048
_N = 32
_X, _Y = 8, 4


def _body(w_ref, h_ref, o_ref):
    o_ref[...] = jnp.sum(w_ref[...] * h_ref[...], axis=2)


@jax.jit
def _run(wT, hT):
    return pl.pallas_call(
        _body,
        out_shape=jax.ShapeDtypeStruct((_X, _Y), jnp.float32),
    )(wT, hT)


def kernel(x, adj, W_att, a_att, W_out):
    hT = jnp.transpose(x[0], (1, 2, 0))
    wT = jnp.transpose(W_out, (1, 2, 0))
    return _run(wT, hT)


# speedup vs baseline: 17.5450x; 1.6876x over previous
import jax
import jax.numpy as jnp
from jax.experimental import pallas as pl

_L = 2048
_X, _Y = 8, 4


def _body(w_ref, h_ref, o_ref):
    s = jnp.sum(w_ref[...] * h_ref[...], axis=2)
    o_ref[...] = s.T


@jax.jit
def _run(wT, hT):
    o48 = pl.pallas_call(
        _body,
        out_shape=jax.ShapeDtypeStruct((_Y, _X), jnp.float32),
    )(wT, hT)
    return jnp.transpose(o48)


def kernel(x, adj, W_att, a_att, W_out):
    hT = jnp.transpose(x[0], (1, 2, 0))
    wT = jnp.transpose(W_out, (1, 2, 0))
    return _run(wT, hT)
